# baseline (speedup 1.0000x reference)
import jax
import jax.numpy as jnp
from jax.experimental import pallas as pl
from jax.experimental.pallas import tpu as pltpu

_TM = 1024
_TK = 512


def _ffn_kernel(x_ref, w1_hbm, b1_ref, w2_hbm, b2_ref, o_ref,
                w1v, w2v, sem1, sem2, *, nk):
    i = pl.program_id(0)
    k = pl.program_id(1)

    @pl.when((i == 0) & (k == 0))
    def _():
        for c in range(nk):
            pltpu.make_async_copy(
                w1_hbm.at[:, pl.ds(c * _TK, _TK)],
                w1v.at[:, pl.ds(c * _TK, _TK)], sem1.at[c]).start()
            pltpu.make_async_copy(
                w2_hbm.at[pl.ds(c * _TK, _TK), :],
                w2v.at[pl.ds(c * _TK, _TK), :], sem2.at[c]).start()

    @pl.when(i == 0)
    def _():
        pltpu.make_async_copy(
            w1_hbm.at[:, pl.ds(k * _TK, _TK)],
            w1v.at[:, pl.ds(k * _TK, _TK)], sem1.at[k]).wait()
        pltpu.make_async_copy(
            w2_hbm.at[pl.ds(k * _TK, _TK), :],
            w2v.at[pl.ds(k * _TK, _TK), :], sem2.at[k]).wait()

    h = jnp.dot(x_ref[...], w1v[:, pl.ds(k * _TK, _TK)],
                preferred_element_type=jnp.float32)
    h = jnp.maximum(h + b1_ref[:, pl.ds(k * _TK, _TK)], 0.0)
    contrib = jnp.dot(h, w2v[pl.ds(k * _TK, _TK), :],
                      preferred_element_type=jnp.float32)

    @pl.when(k == 0)
    def _():
        o_ref[...] = contrib + b2_ref[...]

    @pl.when(k > 0)
    def _():
        o_ref[...] += contrib


def kernel(x, w1, b1, w2, b2):
    B, S, d_model = x.shape
    d_ff = w1.shape[1]
    M = B * S
    nk = d_ff // _TK

    x2d = x.reshape(M, d_model)
    b1_2d = b1.reshape(1, d_ff)
    b2_2d = b2.reshape(1, d_model)

    import functools
    out2d = pl.pallas_call(
        functools.partial(_ffn_kernel, nk=nk),
        out_shape=jax.ShapeDtypeStruct((M, d_model), jnp.float32),
        grid=(M // _TM, nk),
        in_specs=[
            pl.BlockSpec((_TM, d_model), lambda i, k: (i, 0)),
            pl.BlockSpec(memory_space=pltpu.MemorySpace.HBM),
            pl.BlockSpec((1, d_ff), lambda i, k: (0, 0)),
            pl.BlockSpec(memory_space=pltpu.MemorySpace.HBM),
            pl.BlockSpec((1, d_model), lambda i, k: (0, 0)),
        ],
        out_specs=pl.BlockSpec((_TM, d_model), lambda i, k: (i, 0)),
        scratch_shapes=[
            pltpu.VMEM((d_model, d_ff), jnp.float32),
            pltpu.VMEM((d_ff, d_model), jnp.float32),
            pltpu.SemaphoreType.DMA((nk,)),
            pltpu.SemaphoreType.DMA((nk,)),
        ],
        compiler_params=pltpu.CompilerParams(
            dimension_semantics=("arbitrary", "arbitrary"),
            vmem_limit_bytes=60 * 1024 * 1024,
        ),
        cost_estimate=pl.CostEstimate(
            flops=4 * M * d_model * d_ff,
            transcendentals=0,
            bytes_accessed=(x2d.size + w1.size + b1.size + w2.size + b2.size
                            + M * d_model) * 4,
        ),
    )(x2d, w1, b1_2d, w2, b2_2d)

    return out2d.reshape(B, S, d_model)


# speedup vs baseline: 1.1255x; 1.1255x over previous
import jax
import jax.numpy as jnp
from jax.experimental import pallas as pl
from jax.experimental.pallas import tpu as pltpu

_TM = 512
_NC = 8


def _ffn_kernel(x_ref, w1_hbm, b1_ref, w2_hbm, b2_ref, o_ref,
                w1v, w2v, sem1, sem2):
    i = pl.program_id(0)
    d_model, d_ff = w1v.shape
    c1 = d_ff // _NC
    c2 = d_ff // _NC

    def w1_copy(c):
        return pltpu.make_async_copy(
            w1_hbm.at[:, pl.ds(c * c1, c1)],
            w1v.at[:, pl.ds(c * c1, c1)], sem1.at[c])

    def w2_copy(c):
        return pltpu.make_async_copy(
            w2_hbm.at[pl.ds(c * c2, c2), :],
            w2v.at[pl.ds(c * c2, c2), :], sem2.at[c])

    @pl.when(i == 0)
    def _():
        for c in range(_NC):
            w1_copy(c).start()
        for c in range(_NC):
            w2_copy(c).start()
        for c in range(_NC):
            w1_copy(c).wait()

    h = jnp.dot(x_ref[...], w1v[...], preferred_element_type=jnp.float32)
    h = jnp.maximum(h + b1_ref[...], 0.0)

    @pl.when(i == 0)
    def _():
        for c in range(_NC):
            w2_copy(c).wait()

    out = jnp.dot(h, w2v[...], preferred_element_type=jnp.float32)
    o_ref[...] = out + b2_ref[...]


def kernel(x, w1, b1, w2, b2):
    B, S, d_model = x.shape
    d_ff = w1.shape[1]
    M = B * S

    x2d = x.reshape(M, d_model)
    b1_2d = b1.reshape(1, d_ff)
    b2_2d = b2.reshape(1, d_model)

    out2d = pl.pallas_call(
        _ffn_kernel,
        out_shape=jax.ShapeDtypeStruct((M, d_model), jnp.float32),
        grid=(M // _TM,),
        in_specs=[
            pl.BlockSpec((_TM, d_model), lambda i: (i, 0)),
            pl.BlockSpec(memory_space=pltpu.MemorySpace.HBM),
            pl.BlockSpec((1, d_ff), lambda i: (0, 0)),
            pl.BlockSpec(memory_space=pltpu.MemorySpace.HBM),
            pl.BlockSpec((1, d_model), lambda i: (0, 0)),
        ],
        out_specs=pl.BlockSpec((_TM, d_model), lambda i: (i, 0)),
        scratch_shapes=[
            pltpu.VMEM((d_model, d_ff), jnp.float32),
            pltpu.VMEM((d_ff, d_model), jnp.float32),
            pltpu.SemaphoreType.DMA((_NC,)),
            pltpu.SemaphoreType.DMA((_NC,)),
        ],
        compiler_params=pltpu.CompilerParams(
            dimension_semantics=("arbitrary",),
            vmem_limit_bytes=60 * 1024 * 1024,
        ),
        cost_estimate=pl.CostEstimate(
            flops=4 * M * d_model * d_ff,
            transcendentals=0,
            bytes_accessed=(x2d.size + w1.size + b1.size + w2.size + b2.size
                            + M * d_model) * 4,
        ),
    )(x2d, w1, b1_2d, w2, b2_2d)

    return out2d.reshape(B, S, d_model)


# speedup vs baseline: 1.1881x; 1.0556x over previous
import jax
import jax.numpy as jnp
from jax.experimental import pallas as pl
from jax.experimental.pallas import tpu as pltpu

_TM = 512
_NC = 8


def _ffn_kernel(x_ref, w1_hbm, b1_ref, w2_hbm, b2_ref, o_ref,
                w1v, w2v, sem1, sem2):
    i = pl.program_id(0)

    def w1_copy():
        return pltpu.make_async_copy(w1_hbm, w1v, sem1)

    def w2_copy():
        return pltpu.make_async_copy(w2_hbm, w2v, sem2)

    @pl.when(i == 0)
    def _():
        w1_copy().start()
        w2_copy().start()
        w1_copy().wait()

    h = jnp.dot(x_ref[...], w1v[...], preferred_element_type=jnp.float32)
    h = jnp.maximum(h + b1_ref[...], 0.0)

    @pl.when(i == 0)
    def _():
        w2_copy().wait()

    out = jnp.dot(h, w2v[...], preferred_element_type=jnp.float32)
    o_ref[...] = out + b2_ref[...]


def kernel(x, w1, b1, w2, b2):
    B, S, d_model = x.shape
    d_ff = w1.shape[1]
    M = B * S

    x2d = x.reshape(M, d_model)
    b1_2d = b1.reshape(1, d_ff)
    b2_2d = b2.reshape(1, d_model)

    out2d = pl.pallas_call(
        _ffn_kernel,
        out_shape=jax.ShapeDtypeStruct((M, d_model), jnp.float32),
        grid=(M // _TM,),
        in_specs=[
            pl.BlockSpec((_TM, d_model), lambda i: (i, 0)),
            pl.BlockSpec(memory_space=pltpu.MemorySpace.HBM),
            pl.BlockSpec((1, d_ff), lambda i: (0, 0)),
            pl.BlockSpec(memory_space=pltpu.MemorySpace.HBM),
            pl.BlockSpec((1, d_model), lambda i: (0, 0)),
        ],
        out_specs=pl.BlockSpec((_TM, d_model), lambda i: (i, 0)),
        scratch_shapes=[
            pltpu.VMEM((d_model, d_ff), jnp.float32),
            pltpu.VMEM((d_ff, d_model), jnp.float32),
            pltpu.SemaphoreType.DMA,
            pltpu.SemaphoreType.DMA,
        ],
        compiler_params=pltpu.CompilerParams(
            dimension_semantics=("arbitrary",),
            vmem_limit_bytes=60 * 1024 * 1024,
        ),
        cost_estimate=pl.CostEstimate(
            flops=4 * M * d_model * d_ff,
            transcendentals=0,
            bytes_accessed=(x2d.size + w1.size + b1.size + w2.size + b2.size
                            + M * d_model) * 4,
        ),
    )(x2d, w1, b1_2d, w2, b2_2d)

    return out2d.reshape(B, S, d_model)
